# baseline (speedup 1.0000x reference)
import functools

import jax
import jax.numpy as jnp
from jax import lax
from jax.experimental import pallas as pl
from jax.experimental.pallas import tpu as pltpu

_IN_CH = 512
_CH = 512
_EMBED_DIM = 65920
_NUM_EMB = 1024
_NUM_CB = 8
_DIM = _EMBED_DIM // _NUM_CB
_DEC_IN = 128
_DEC_H = 256
_DATA_Y = 128

_NC = 5
_CD = _DIM // _NC


def _dot_t(a, w, precision=lax.Precision.HIGHEST):
    return lax.dot_general(
        a, w, (((1,), (1,)), ((), ())),
        preferred_element_type=jnp.float32, precision=precision)


_NB = _NUM_CB * _NC
_NT = _NB + 1


def _vq_kernel(x_r, wf_r, bf_r, wx1_r, bx1_r, wx2_r, bx2_r,
               wd1_r, bd1_r, wd2_r, bd2_r, wc_r, bc_r, emb_r,
               dec_o, diffs_o, perp_o, pre_ref, z_ref, acc_ref, zsq_ref):
    s = pl.program_id(0)

    @pl.when(s == 0)
    def _init():
        pre_ref[...] = jnp.maximum(_dot_t(x_r[...], wf_r[...]) + bf_r[...], 0.0)
        diffs_o[...] = jnp.zeros((1, 1), jnp.float32)

    @pl.when(s >= 1)
    def _dist():
        t = s - 1
        c = t % _NC
        emb = emb_r[0, 0]
        z = z_ref[...]
        emb_bf = emb.astype(jnp.bfloat16)
        cross = lax.dot_general(
            z.astype(jnp.bfloat16), emb_bf, (((1,), (0,)), ((), ())),
            preferred_element_type=jnp.float32)
        esq = jnp.sum(emb * emb, axis=0, keepdims=True)
        contrib = esq - 2.0 * cross
        zsq = jnp.sum(z * z).reshape(1, 1)

        @pl.when(c == 0)
        def _first():
            acc_ref[...] = contrib
            zsq_ref[...] = zsq

        @pl.when(c > 0)
        def _rest():
            acc_ref[...] = acc_ref[...] + contrib
            zsq_ref[...] = zsq_ref[...] + zsq

        @pl.when(c == _NC - 1)
        def _finish_cb():
            dist = acc_ref[...] + zsq_ref[...]
            m = jnp.min(dist)
            diffs_o[...] = diffs_o[...] + jnp.full((1, 1), m / _DIM,
                                                   jnp.float32)

    @pl.when(s <= _NB - 1)
    def _znext():
        wc = wc_r[0, 0]
        bc = bc_r[0, 0]
        z_ref[...] = lax.dot_general(
            pre_ref[...].astype(jnp.bfloat16), wc.astype(jnp.bfloat16),
            (((1,), (1,)), ((), ())),
            preferred_element_type=jnp.float32) + bc

    @pl.when(s == _NT - 1)
    def _tail():
        xv = x_r[...]
        e1 = jnp.maximum(_dot_t(xv, wx1_r[...]) + bx1_r[...], 0.0)
        e2 = _dot_t(e1, wx2_r[...]) + bx2_r[...]
        d1 = jnp.maximum(_dot_t(e2, wd1_r[...]) + bd1_r[...], 0.0)
        dec_o[...] = _dot_t(d1, wd2_r[...]) + bd2_r[...]
        lg = jnp.log(jnp.float32(1.0) + jnp.float32(1e-10))
        perp_o[...] = jnp.full((1, 1), jnp.exp(-jnp.float32(_NUM_CB) * lg),
                               jnp.float32)


def _wc_map(s):
    t = jnp.minimum(s, _NB - 1)
    return (t // _NC, t % _NC, 0, 0)


def _emb_map(s):
    t = jnp.maximum(s - 1, 0)
    return (t // _NC, t % _NC, 0, 0)


@functools.partial(jax.jit, static_argnums=())
def kernel(x, Wf, bf, Wx1, bx1, Wx2, bx2, Wc, bc, Wd1, bd1, Wd2, bd2, embeds):
    wc4 = Wc.reshape(_NUM_CB, _NC, _CD, _CH)
    bc4 = bc.reshape(_NUM_CB, _NC, 1, _CD)
    emb4 = embeds.reshape(_NUM_CB, _NC, _CD, _NUM_EMB)

    const2 = lambda shape: pl.BlockSpec(shape, lambda s: (0, 0))
    dec, diffs, perp = pl.pallas_call(
        _vq_kernel,
        grid=(_NT,),
        in_specs=[
            const2((1, _IN_CH)),
            const2((_CH, _IN_CH)),
            const2((1, _CH)),
            const2((_CH, _IN_CH)),
            const2((1, _CH)),
            const2((_DEC_IN, _CH)),
            const2((1, _DEC_IN)),
            const2((_DEC_H, _DEC_IN)),
            const2((1, _DEC_H)),
            const2((_DATA_Y, _DEC_H)),
            const2((1, _DATA_Y)),
            pl.BlockSpec((1, 1, _CD, _CH), _wc_map),
            pl.BlockSpec((1, 1, 1, _CD), _wc_map),
            pl.BlockSpec((1, 1, _CD, _NUM_EMB), _emb_map),
        ],
        out_specs=[
            pl.BlockSpec((1, _DATA_Y), lambda s: (0, 0)),
            pl.BlockSpec((1, 1), lambda s: (0, 0)),
            pl.BlockSpec((1, 1), lambda s: (0, 0)),
        ],
        out_shape=[
            jax.ShapeDtypeStruct((1, _DATA_Y), jnp.float32),
            jax.ShapeDtypeStruct((1, 1), jnp.float32),
            jax.ShapeDtypeStruct((1, 1), jnp.float32),
        ],
        scratch_shapes=[
            pltpu.VMEM((1, _CH), jnp.float32),
            pltpu.VMEM((1, _CD), jnp.float32),
            pltpu.VMEM((1, _NUM_EMB), jnp.float32),
            pltpu.VMEM((1, 1), jnp.float32),
        ],
    )(x, Wf, bf.reshape(1, _CH), Wx1, bx1.reshape(1, _CH),
      Wx2, bx2.reshape(1, _DEC_IN), Wd1, bd1.reshape(1, _DEC_H),
      Wd2, bd2.reshape(1, _DATA_Y), wc4, bc4, emb4)

    return dec, diffs.reshape(()), perp.reshape(())


# speedup vs baseline: 1.0383x; 1.0383x over previous
import functools

import jax
import jax.numpy as jnp
from jax import lax
from jax.experimental import pallas as pl
from jax.experimental.pallas import tpu as pltpu

_IN_CH = 512
_CH = 512
_EMBED_DIM = 65920
_NUM_EMB = 1024
_NUM_CB = 8
_DIM = _EMBED_DIM // _NUM_CB
_DEC_IN = 128
_DEC_H = 256
_DATA_Y = 128

_NC = 5
_CD = _DIM // _NC


def _dot_t(a, w, precision=lax.Precision.HIGHEST):
    return lax.dot_general(
        a, w, (((1,), (1,)), ((), ())),
        preferred_element_type=jnp.float32, precision=precision)


def _vq_kernel(x_r, wf_r, bf_r, wx1_r, bx1_r, wx2_r, bx2_r,
               wd1_r, bd1_r, wd2_r, bd2_r, wc_r, bc_r, emba_r, embb_r,
               dec_o, diffs_o, perp_o, pre_ref, acc_ref, zsq_ref):
    k = pl.program_id(0)
    c = pl.program_id(1)

    @pl.when((k == 0) & (c == 0))
    def _init():
        pre_ref[...] = jnp.maximum(_dot_t(x_r[...], wf_r[...]) + bf_r[...], 0.0)
        diffs_o[...] = jnp.zeros((1, 1), jnp.float32)

    @pl.when((k == _NUM_CB - 1) & (c == _NC - 1))
    def _tail():
        xv = x_r[...]
        e1 = jnp.maximum(_dot_t(xv, wx1_r[...]) + bx1_r[...], 0.0)
        e2 = _dot_t(e1, wx2_r[...]) + bx2_r[...]
        d1 = jnp.maximum(_dot_t(e2, wd1_r[...]) + bd1_r[...], 0.0)
        dec_o[...] = _dot_t(d1, wd2_r[...]) + bd2_r[...]
        lg = jnp.log(jnp.float32(1.0) + jnp.float32(1e-10))
        perp_o[...] = jnp.full((1, 1), jnp.exp(-jnp.float32(_NUM_CB) * lg),
                               jnp.float32)

    wc = wc_r[0, 0]
    bc = bc_r[0, 0]

    z = lax.dot_general(
        pre_ref[...].astype(jnp.bfloat16), wc.astype(jnp.bfloat16),
        (((1,), (1,)), ((), ())),
        preferred_element_type=jnp.float32) + bc
    z_bf = z.astype(jnp.bfloat16)

    def _half(e_r):
        emb = e_r[0, 0]
        cr = lax.dot_general(
            z_bf, emb.astype(jnp.bfloat16), (((1,), (0,)), ((), ())),
            preferred_element_type=jnp.float32)
        esq = jnp.sum(emb * emb, axis=0, keepdims=True)
        return esq - 2.0 * cr

    contrib = jnp.concatenate([_half(emba_r), _half(embb_r)], axis=1)
    zsq = jnp.sum(z * z).reshape(1, 1)

    @pl.when(c == 0)
    def _first():
        acc_ref[...] = contrib
        zsq_ref[...] = zsq

    @pl.when(c > 0)
    def _rest():
        acc_ref[...] = acc_ref[...] + contrib
        zsq_ref[...] = zsq_ref[...] + zsq

    @pl.when(c == _NC - 1)
    def _finish_cb():
        dist = acc_ref[...] + zsq_ref[...]
        m = jnp.min(dist)
        diffs_o[...] = diffs_o[...] + jnp.full((1, 1), m / _DIM, jnp.float32)


@functools.partial(jax.jit, static_argnums=())
def kernel(x, Wf, bf, Wx1, bx1, Wx2, bx2, Wc, bc, Wd1, bd1, Wd2, bd2, embeds):
    wc4 = Wc.reshape(_NUM_CB, _NC, _CD, _CH)
    bc4 = bc.reshape(_NUM_CB, _NC, 1, _CD)
    emb4 = embeds.reshape(_NUM_CB, _NC, _CD, _NUM_EMB)

    const2 = lambda shape: pl.BlockSpec(shape, lambda k, c: (0, 0))
    dec, diffs, perp = pl.pallas_call(
        _vq_kernel,
        grid=(_NUM_CB, _NC),
        in_specs=[
            const2((1, _IN_CH)),
            const2((_CH, _IN_CH)),
            const2((1, _CH)),
            const2((_CH, _IN_CH)),
            const2((1, _CH)),
            const2((_DEC_IN, _CH)),
            const2((1, _DEC_IN)),
            const2((_DEC_H, _DEC_IN)),
            const2((1, _DEC_H)),
            const2((_DATA_Y, _DEC_H)),
            const2((1, _DATA_Y)),
            pl.BlockSpec((1, 1, _CD, _CH), lambda k, c: (k, c, 0, 0)),
            pl.BlockSpec((1, 1, 1, _CD), lambda k, c: (k, c, 0, 0)),
            pl.BlockSpec((1, 1, _CD, _NUM_EMB // 2), lambda k, c: (k, c, 0, 0)),
            pl.BlockSpec((1, 1, _CD, _NUM_EMB // 2), lambda k, c: (k, c, 0, 1)),
        ],
        out_specs=[
            pl.BlockSpec((1, _DATA_Y), lambda k, c: (0, 0)),
            pl.BlockSpec((1, 1), lambda k, c: (0, 0)),
            pl.BlockSpec((1, 1), lambda k, c: (0, 0)),
        ],
        out_shape=[
            jax.ShapeDtypeStruct((1, _DATA_Y), jnp.float32),
            jax.ShapeDtypeStruct((1, 1), jnp.float32),
            jax.ShapeDtypeStruct((1, 1), jnp.float32),
        ],
        scratch_shapes=[
            pltpu.VMEM((1, _CH), jnp.float32),
            pltpu.VMEM((1, _NUM_EMB), jnp.float32),
            pltpu.VMEM((1, 1), jnp.float32),
        ],
    )(x, Wf, bf.reshape(1, _CH), Wx1, bx1.reshape(1, _CH),
      Wx2, bx2.reshape(1, _DEC_IN), Wd1, bd1.reshape(1, _DEC_H),
      Wd2, bd2.reshape(1, _DATA_Y), wc4, bc4, emb4, emb4)

    return dec, diffs.reshape(()), perp.reshape(())


# speedup vs baseline: 1.0481x; 1.0094x over previous
import functools

import jax
import jax.numpy as jnp
from jax import lax
from jax.experimental import pallas as pl
from jax.experimental.pallas import tpu as pltpu

_IN_CH = 512
_CH = 512
_EMBED_DIM = 65920
_NUM_EMB = 1024
_NUM_CB = 8
_DIM = _EMBED_DIM // _NUM_CB
_DEC_IN = 128
_DEC_H = 256
_DATA_Y = 128

_NC = 5
_CD = _DIM // _NC


def _dot_t(a, w, precision=lax.Precision.HIGHEST):
    return lax.dot_general(
        a, w, (((1,), (1,)), ((), ())),
        preferred_element_type=jnp.float32, precision=precision)


def _vq_kernel(x_r, wf_r, bf_r, wx1_r, bx1_r, wx2_r, bx2_r,
               wd1_r, bd1_r, wd2_r, bd2_r, wc_r, bc_r, emb_r,
               dec_o, diffs_o, perp_o, pre_ref, acc_ref, zsq_ref):
    k = pl.program_id(0)
    c = pl.program_id(1)

    @pl.when((k == 0) & (c == 0))
    def _init():
        pre_ref[...] = jnp.maximum(_dot_t(x_r[...], wf_r[...]) + bf_r[...], 0.0)
        diffs_o[...] = jnp.zeros((1, 1), jnp.float32)

    @pl.when((k == _NUM_CB - 1) & (c == _NC - 1))
    def _tail():
        xv = x_r[...]
        e1 = jnp.maximum(_dot_t(xv, wx1_r[...]) + bx1_r[...], 0.0)
        e2 = _dot_t(e1, wx2_r[...]) + bx2_r[...]
        d1 = jnp.maximum(_dot_t(e2, wd1_r[...]) + bd1_r[...], 0.0)
        dec_o[...] = _dot_t(d1, wd2_r[...]) + bd2_r[...]
        lg = jnp.log(jnp.float32(1.0) + jnp.float32(1e-10))
        perp_o[...] = jnp.full((1, 1), jnp.exp(-jnp.float32(_NUM_CB) * lg),
                               jnp.float32)

    wc = wc_r[0, 0]
    emb = emb_r[0, 0]
    bc = bc_r[0, 0]

    z = lax.dot_general(
        pre_ref[...].astype(jnp.bfloat16), wc.astype(jnp.bfloat16),
        (((1,), (1,)), ((), ())),
        preferred_element_type=jnp.float32) + bc
    emb_bf = emb.astype(jnp.bfloat16)
    cross = lax.dot_general(
        z.astype(jnp.bfloat16), emb_bf, (((1,), (0,)), ((), ())),
        preferred_element_type=jnp.float32)
    esq = jnp.sum(emb * emb, axis=0, keepdims=True)
    contrib = esq - 2.0 * cross
    zsq = jnp.sum(z * z).reshape(1, 1)

    @pl.when(c == 0)
    def _first():
        acc_ref[...] = contrib
        zsq_ref[...] = zsq

    @pl.when(c > 0)
    def _rest():
        acc_ref[...] = acc_ref[...] + contrib
        zsq_ref[...] = zsq_ref[...] + zsq

    @pl.when(c == _NC - 1)
    def _finish_cb():
        dist = acc_ref[...] + zsq_ref[...]
        m = jnp.min(dist)
        diffs_o[...] = diffs_o[...] + jnp.full((1, 1), m / _DIM, jnp.float32)


@functools.partial(jax.jit, static_argnums=())
def kernel(x, Wf, bf, Wx1, bx1, Wx2, bx2, Wc, bc, Wd1, bd1, Wd2, bd2, embeds):
    wc4 = Wc.reshape(_NUM_CB, _NC, _CD, _CH)
    bc4 = bc.reshape(_NUM_CB, _NC, 1, _CD)
    emb4 = embeds.reshape(_NUM_CB, _NC, _CD, _NUM_EMB)

    const2 = lambda shape: pl.BlockSpec(shape, lambda k, c: (0, 0))
    dec, diffs, perp = pl.pallas_call(
        _vq_kernel,
        grid=(_NUM_CB, _NC),
        in_specs=[
            const2((1, _IN_CH)),
            const2((_CH, _IN_CH)),
            const2((1, _CH)),
            const2((_CH, _IN_CH)),
            const2((1, _CH)),
            const2((_DEC_IN, _CH)),
            const2((1, _DEC_IN)),
            const2((_DEC_H, _DEC_IN)),
            const2((1, _DEC_H)),
            const2((_DATA_Y, _DEC_H)),
            const2((1, _DATA_Y)),
            pl.BlockSpec((1, 1, _CD, _CH), lambda k, c: (k, c, 0, 0)),
            pl.BlockSpec((1, 1, 1, _CD), lambda k, c: (k, c, 0, 0)),
            pl.BlockSpec((1, 1, _CD, _NUM_EMB), lambda k, c: (k, c, 0, 0)),
        ],
        out_specs=[
            pl.BlockSpec((1, _DATA_Y), lambda k, c: (0, 0)),
            pl.BlockSpec((1, 1), lambda k, c: (0, 0)),
            pl.BlockSpec((1, 1), lambda k, c: (0, 0)),
        ],
        out_shape=[
            jax.ShapeDtypeStruct((1, _DATA_Y), jnp.float32),
            jax.ShapeDtypeStruct((1, 1), jnp.float32),
            jax.ShapeDtypeStruct((1, 1), jnp.float32),
        ],
        scratch_shapes=[
            pltpu.VMEM((1, _CH), jnp.float32),
            pltpu.VMEM((1, _NUM_EMB), jnp.float32),
            pltpu.VMEM((1, 1), jnp.float32),
        ],
    )(x, Wf, bf.reshape(1, _CH), Wx1, bx1.reshape(1, _CH),
      Wx2, bx2.reshape(1, _DEC_IN), Wd1, bd1.reshape(1, _DEC_H),
      Wd2, bd2.reshape(1, _DATA_Y), wc4, bc4, emb4)

    return dec, diffs.reshape(()), perp.reshape(())
